# baseline (speedup 1.0000x reference)
import functools

import jax
import jax.numpy as jnp
from jax import lax
from jax.experimental import pallas as pl
from jax.experimental.pallas import tpu as pltpu
from jax.experimental.pallas import tpu_sc as plsc

NC = 2
NS = 16
CH = 128
NB = 2

N = 10000
E = 160000
N_ACC = 10240
RPT = N_ACC // NS
E_PAD = 163840
CHUNKS1 = E_PAD // (NS * CH)
CHUNKS2 = E_PAD // (NC * NS * CH)

BN = 2000
GRID = N // BN


def _sc_mesh():
    return plsc.VectorSubcoreMesh(core_axis_name="c", subcore_axis_name="s")


def _deg_body(dst_hbm, zd_hbm, out_hbm, didx_v, ones_v, dacc_sh):
    c = lax.axis_index("c")
    s = lax.axis_index("s")
    for i in range(CH // 16):
        ones_v[pl.ds(i * 16, 16)] = jnp.ones((16,), jnp.float32)
    pltpu.sync_copy(zd_hbm.at[pl.ds(s * RPT, RPT)], dacc_sh.at[pl.ds(s * RPT, RPT)])
    base = (c * NS + s) * (CHUNKS2 * CH)
    pltpu.sync_copy(dst_hbm.at[pl.ds(base, CHUNKS2 * CH)], didx_v)
    plsc.subcore_barrier()

    def body(k, carry):
        pltpu.sync_copy(ones_v, dacc_sh.at[didx_v.at[pl.ds(k * CH, CH)]],
                        add=True)
        return carry

    lax.fori_loop(0, CHUNKS2, body, 0)
    plsc.subcore_barrier()
    pltpu.sync_copy(dacc_sh.at[pl.ds(s * RPT, RPT)], out_hbm.at[c, pl.ds(s * RPT, RPT)])


def _deg_call(dst_p, zerosd):
    f = pl.kernel(
        _deg_body,
        out_type=jax.ShapeDtypeStruct((NC, N_ACC), jnp.float32),
        mesh=_sc_mesh(),
        scratch_types=[
            pltpu.VMEM((CHUNKS2 * CH,), jnp.int32),
            pltpu.VMEM((CH,), jnp.float32),
            pltpu.VMEM_SHARED((N_ACC,), jnp.float32),
        ],
    )
    return f(dst_p, zerosd)


def _agg_body(feat_split, src_hbm, dst_hbm, tab_hbm, z_hbm, out_hbm,
              gidx_v, didx_v, rows_v, sems, dacc_sh):
    c = lax.axis_index("c")
    s = lax.axis_index("s")
    pltpu.sync_copy(z_hbm.at[pl.ds(s * RPT, RPT)], dacc_sh.at[pl.ds(s * RPT, RPT)])
    if feat_split:
        base = s * (CHUNKS1 * CH)
        nchunks = CHUNKS1
        nphase = 2
    else:
        base = (c * NS + s) * (CHUNKS2 * CH)
        nchunks = CHUNKS2
        nphase = 1
    src_row = c
    pch = nchunks // nphase

    def start(k, b):
        pltpu.async_copy(tab_hbm.at[gidx_v.at[pl.ds(k * CH, CH)]],
                         rows_v[b], sems[b])

    for p in range(nphase):
        off = base + p * (pch * CH)
        pltpu.sync_copy(src_hbm.at[src_row, pl.ds(off, pch * CH)], gidx_v)
        pltpu.sync_copy(dst_hbm.at[pl.ds(off, pch * CH)], didx_v)
        if p == 0:
            plsc.subcore_barrier()

        for j in range(NB - 1):
            start(j, j)

        def body(i, carry):
            for j in range(NB):
                k = NB * i + j

                @pl.when(k + NB - 1 < pch)
                def _():
                    start(k + NB - 1, (j + NB - 1) % NB)

                pltpu.make_async_copy(tab_hbm.at[gidx_v.at[pl.ds(k * CH, CH)]],
                                      rows_v[j], sems[j]).wait()
                pltpu.sync_copy(rows_v[j],
                                dacc_sh.at[didx_v.at[pl.ds(k * CH, CH)]],
                                add=True)
            return carry

        lax.fori_loop(0, pch // NB, body, 0)

    plsc.subcore_barrier()
    pltpu.sync_copy(dacc_sh.at[pl.ds(s * RPT, RPT)],
                    out_hbm.at[c, pl.ds(s * RPT, RPT)])


def _agg_call(src2, dst_p, table, zeros, feat_split):
    npre = (CHUNKS1 // 2 if feat_split else CHUNKS2) * CH
    f = pl.kernel(
        functools.partial(_agg_body, feat_split),
        out_type=jax.ShapeDtypeStruct((NC, N_ACC, 128), jnp.float32),
        mesh=_sc_mesh(),
        scratch_types=[
            pltpu.VMEM((npre,), jnp.int32),
            pltpu.VMEM((npre,), jnp.int32),
            [pltpu.VMEM((CH, 128), jnp.float32) for _ in range(NB)],
            [pltpu.SemaphoreType.DMA for _ in range(NB)],
            pltpu.VMEM_SHARED((N_ACC, 128), jnp.float32),
        ],
    )
    return f(src2, dst_p, table, zeros)


def _b_body(x_ref, w_ref, dg_ref, hs_ref, dis_ref):
    deg = dg_ref[:, 0:1] + dg_ref[:, 1:2] + 1.0
    dis = lax.rsqrt(deg)
    h = jnp.dot(x_ref[...], w_ref[...], preferred_element_type=jnp.float32)
    hs = h * dis
    hs_ref[0] = hs[:, :128]
    hs_ref[1] = hs[:, 128:]
    dis_ref[...] = dis


def _b_call(x, W1, degt):
    din = x.shape[1]
    return pl.pallas_call(
        _b_body,
        grid=(GRID,),
        in_specs=[
            pl.BlockSpec((BN, din), lambda i: (i, 0)),
            pl.BlockSpec((din, 256), lambda i: (0, 0)),
            pl.BlockSpec((BN, 2), lambda i: (i, 0)),
        ],
        out_specs=[
            pl.BlockSpec((2, BN, 128), lambda i: (0, i, 0)),
            pl.BlockSpec((BN, 1), lambda i: (i, 0)),
        ],
        out_shape=[
            jax.ShapeDtypeStruct((2, N, 128), jnp.float32),
            jax.ShapeDtypeStruct((N, 1), jnp.float32),
        ],
    )(x, W1, degt)


def _d_body(aa_ref, ab_ref, ha_ref, hb_ref, dis_ref, b1_ref, w2_ref, out_ref):
    t = jnp.concatenate([aa_ref[0] + ha_ref[0], ab_ref[0] + hb_ref[0]], axis=1)
    z = t * dis_ref[...] + b1_ref[...]
    r = jnp.maximum(z, 0.0)
    h2 = jnp.dot(r, w2_ref[...], preferred_element_type=jnp.float32)
    hs2 = h2 * dis_ref[...]
    padded = jnp.concatenate(
        [hs2, jnp.zeros((hs2.shape[0], 128 - hs2.shape[1]), jnp.float32)], axis=1)
    out_ref[0] = padded
    out_ref[1] = padded


def _d_call(acc1, hs2n, dis, b1, W2):
    dout = W2.shape[1]
    return pl.pallas_call(
        _d_body,
        grid=(GRID,),
        in_specs=[
            pl.BlockSpec((1, BN, 128), lambda i: (0, i, 0)),
            pl.BlockSpec((1, BN, 128), lambda i: (1, i, 0)),
            pl.BlockSpec((1, BN, 128), lambda i: (0, i, 0)),
            pl.BlockSpec((1, BN, 128), lambda i: (1, i, 0)),
            pl.BlockSpec((BN, 1), lambda i: (i, 0)),
            pl.BlockSpec((1, 256), lambda i: (0, 0)),
            pl.BlockSpec((256, dout), lambda i: (0, 0)),
        ],
        out_specs=pl.BlockSpec((2, BN, 128), lambda i: (0, i, 0)),
        out_shape=jax.ShapeDtypeStruct((2, N, 128), jnp.float32),
    )(acc1, acc1, hs2n, hs2n, dis, b1, W2)


def _f_body(aa_ref, ab_ref, hs_ref, dis_ref, b2_ref, out_ref):
    dout = b2_ref.shape[1]
    z = ((aa_ref[0, :, :dout] + ab_ref[0, :, :dout] + hs_ref[0, :, :dout])
         * dis_ref[...] + b2_ref[...])
    m = jnp.max(z, axis=1, keepdims=True)
    ez = jnp.exp(z - m)
    lse = jnp.log(jnp.sum(ez, axis=1, keepdims=True))
    out_ref[...] = z - m - lse


def _f_call(acc2, hs2, dis, b2):
    dout = b2.shape[1]
    return pl.pallas_call(
        _f_body,
        grid=(GRID,),
        in_specs=[
            pl.BlockSpec((1, BN, 128), lambda i: (0, i, 0)),
            pl.BlockSpec((1, BN, 128), lambda i: (1, i, 0)),
            pl.BlockSpec((1, BN, 128), lambda i: (0, i, 0)),
            pl.BlockSpec((BN, 1), lambda i: (i, 0)),
            pl.BlockSpec((1, dout), lambda i: (0, 0)),
        ],
        out_specs=pl.BlockSpec((BN, dout), lambda i: (i, 0)),
        out_shape=jax.ShapeDtypeStruct((N, dout), jnp.float32),
    )(acc2, acc2, hs2, dis, b2)


def kernel(x, edge_index, W1, b1, W2, b2):
    src = edge_index[0]
    dst = edge_index[1]
    pad = E_PAD - E
    src_p = jnp.concatenate([src, jnp.zeros((pad,), jnp.int32)])
    dump = N + jnp.arange(pad, dtype=jnp.int32) % (N_ACC - N)
    dst_p = jnp.concatenate([dst, dump])
    src2 = jnp.stack([src_p, src_p + N])

    zerosd = jnp.zeros((N_ACC,), jnp.float32)
    zeros1 = jnp.zeros((N_ACC, 128), jnp.float32)

    degout = _deg_call(dst_p, zerosd)
    degt = degout.T

    hs2n, dis = _b_call(x, W1, degt)
    tab1 = hs2n.reshape(2 * N, 128)
    acc1 = _agg_call(src2, dst_p, tab1, zeros1, True)

    hs2 = _d_call(acc1, hs2n, dis, b1.reshape(1, -1), W2)
    tab2 = hs2.reshape(2 * N, 128)
    acc2 = _agg_call(src2, dst_p, tab2, zeros1, False)

    return _f_call(acc2, hs2, dis, b2.reshape(1, -1))


# speedup vs baseline: 2.6354x; 2.6354x over previous
import functools

import jax
import jax.numpy as jnp
from jax import lax
from jax.experimental import pallas as pl
from jax.experimental.pallas import tpu as pltpu
from jax.experimental.pallas import tpu_sc as plsc

NC = 2
NS = 16
CH = 128
NB = 2

N = 10000
E = 160000
N_ACC = 10240
RPT = N_ACC // NS
E_PAD = 163840
CHUNKS1 = E_PAD // (NS * CH)
CHUNKS2 = E_PAD // (NC * NS * CH)

BN = 2000
GRID = N // BN


def _sc_mesh():
    return plsc.VectorSubcoreMesh(core_axis_name="c", subcore_axis_name="s")


def _deg_body(dst_hbm, zd_hbm, out_hbm, didx_v, ones_v, dacc_sh):
    c = lax.axis_index("c")
    s = lax.axis_index("s")
    for i in range(CH // 16):
        ones_v[pl.ds(i * 16, 16)] = jnp.ones((16,), jnp.float32)
    pltpu.sync_copy(zd_hbm.at[pl.ds(s * RPT, RPT)], dacc_sh.at[pl.ds(s * RPT, RPT)])
    base = (c * NS + s) * (CHUNKS2 * CH)
    pltpu.sync_copy(dst_hbm.at[pl.ds(base, CHUNKS2 * CH)], didx_v)
    plsc.subcore_barrier()

    def body(k, carry):
        pltpu.sync_copy(ones_v, dacc_sh.at[didx_v.at[pl.ds(k * CH, CH)]],
                        add=True)
        return carry

    lax.fori_loop(0, CHUNKS2, body, 0)
    plsc.subcore_barrier()
    pltpu.sync_copy(dacc_sh.at[pl.ds(s * RPT, RPT)], out_hbm.at[c, pl.ds(s * RPT, RPT)])


def _deg_call(dst_p, zerosd):
    f = pl.kernel(
        _deg_body,
        out_type=jax.ShapeDtypeStruct((NC, N_ACC), jnp.float32),
        mesh=_sc_mesh(),
        scratch_types=[
            pltpu.VMEM((CHUNKS2 * CH,), jnp.int32),
            pltpu.VMEM((CH,), jnp.float32),
            pltpu.VMEM_SHARED((N_ACC,), jnp.float32),
        ],
    )
    return f(dst_p, zerosd)


def _agg_body(feat_split, src_hbm, dst_hbm, tab_hbm, z_hbm, out_hbm,
              gidx_v, didx_v, rows_v, sems, dacc_sh):
    c = lax.axis_index("c")
    s = lax.axis_index("s")
    pltpu.sync_copy(z_hbm.at[pl.ds(s * RPT, RPT)], dacc_sh.at[pl.ds(s * RPT, RPT)])
    if feat_split:
        base = s * (CHUNKS1 * CH)
        nchunks = CHUNKS1
        src_row = c
        nphase = 2
    else:
        base = (c * NS + s) * (CHUNKS2 * CH)
        nchunks = CHUNKS2
        nphase = 1
        src_row = 0
    pch = nchunks // nphase

    def start(k, b):
        pltpu.async_copy(tab_hbm.at[gidx_v.at[pl.ds(k * CH, CH)]],
                         rows_v[b], sems[b])

    for p in range(nphase):
        off = base + p * (pch * CH)
        pltpu.sync_copy(src_hbm.at[src_row, pl.ds(off, pch * CH)], gidx_v)
        pltpu.sync_copy(dst_hbm.at[pl.ds(off, pch * CH)], didx_v)
        if p == 0:
            plsc.subcore_barrier()

        for j in range(NB - 1):
            start(j, j)

        def body(i, carry):
            for j in range(NB):
                k = NB * i + j

                @pl.when(k + NB - 1 < pch)
                def _():
                    start(k + NB - 1, (j + NB - 1) % NB)

                pltpu.make_async_copy(tab_hbm.at[gidx_v.at[pl.ds(k * CH, CH)]],
                                      rows_v[j], sems[j]).wait()
                pltpu.sync_copy(rows_v[j],
                                dacc_sh.at[didx_v.at[pl.ds(k * CH, CH)]],
                                add=True)
            return carry

        lax.fori_loop(0, pch // NB, body, 0)

    plsc.subcore_barrier()
    pltpu.sync_copy(dacc_sh.at[pl.ds(s * RPT, RPT)],
                    out_hbm.at[c, pl.ds(s * RPT, RPT)])


def _agg_call(src2, dst_p, table, zeros, feat_split):
    npre = (CHUNKS1 // 2 if feat_split else CHUNKS2) * CH
    f = pl.kernel(
        functools.partial(_agg_body, feat_split),
        out_type=jax.ShapeDtypeStruct((NC, N_ACC, 128), jnp.float32),
        mesh=_sc_mesh(),
        scratch_types=[
            pltpu.VMEM((npre,), jnp.int32),
            pltpu.VMEM((npre,), jnp.int32),
            [pltpu.VMEM((CH, 128), jnp.float32) for _ in range(NB)],
            [pltpu.SemaphoreType.DMA for _ in range(NB)],
            pltpu.VMEM_SHARED((N_ACC, 128), jnp.float32),
        ],
    )
    return f(src2, dst_p, table, zeros)


def _b_body(x_ref, w_ref, dg_ref, hs_ref, dis_ref):
    deg = dg_ref[:, 0:1] + dg_ref[:, 1:2] + 1.0
    dis = lax.rsqrt(deg)
    h = jnp.dot(x_ref[...], w_ref[...], preferred_element_type=jnp.float32)
    hs = h * dis
    hs_ref[0] = hs[:, :128]
    hs_ref[1] = hs[:, 128:]
    dis_ref[...] = dis


def _b_call(x, W1, degt):
    din = x.shape[1]
    return pl.pallas_call(
        _b_body,
        grid=(GRID,),
        in_specs=[
            pl.BlockSpec((BN, din), lambda i: (i, 0)),
            pl.BlockSpec((din, 256), lambda i: (0, 0)),
            pl.BlockSpec((BN, 2), lambda i: (i, 0)),
        ],
        out_specs=[
            pl.BlockSpec((2, BN, 128), lambda i: (0, i, 0)),
            pl.BlockSpec((BN, 1), lambda i: (i, 0)),
        ],
        out_shape=[
            jax.ShapeDtypeStruct((2, N, 128), jnp.float32),
            jax.ShapeDtypeStruct((N, 1), jnp.float32),
        ],
    )(x, W1, degt)


def _d_body(aa_ref, ab_ref, ha_ref, hb_ref, dis_ref, b1_ref, w2_ref, out_ref):
    t = jnp.concatenate([aa_ref[0] + ha_ref[0], ab_ref[0] + hb_ref[0]], axis=1)
    z = t * dis_ref[...] + b1_ref[...]
    r = jnp.maximum(z, 0.0)
    h2 = jnp.dot(r, w2_ref[...], preferred_element_type=jnp.float32)
    hs2 = h2 * dis_ref[...]
    out_ref[...] = jnp.concatenate(
        [hs2, jnp.zeros((hs2.shape[0], 128 - hs2.shape[1]), jnp.float32)], axis=1)


def _d_call(acc1, hs2n, dis, b1, W2):
    dout = W2.shape[1]
    return pl.pallas_call(
        _d_body,
        grid=(GRID,),
        in_specs=[
            pl.BlockSpec((1, BN, 128), lambda i: (0, i, 0)),
            pl.BlockSpec((1, BN, 128), lambda i: (1, i, 0)),
            pl.BlockSpec((1, BN, 128), lambda i: (0, i, 0)),
            pl.BlockSpec((1, BN, 128), lambda i: (1, i, 0)),
            pl.BlockSpec((BN, 1), lambda i: (i, 0)),
            pl.BlockSpec((1, 256), lambda i: (0, 0)),
            pl.BlockSpec((256, dout), lambda i: (0, 0)),
        ],
        out_specs=pl.BlockSpec((BN, 128), lambda i: (i, 0)),
        out_shape=jax.ShapeDtypeStruct((N, 128), jnp.float32),
    )(acc1, acc1, hs2n, hs2n, dis, b1, W2)


def _f_body(aa_ref, ab_ref, hs_ref, dis_ref, b2_ref, out_ref):
    dout = b2_ref.shape[1]
    z = ((aa_ref[0, :, :dout] + ab_ref[0, :, :dout] + hs_ref[:, :dout])
         * dis_ref[...] + b2_ref[...])
    m = jnp.max(z, axis=1, keepdims=True)
    ez = jnp.exp(z - m)
    lse = jnp.log(jnp.sum(ez, axis=1, keepdims=True))
    out_ref[...] = z - m - lse


def _f_call(acc2, hs2, dis, b2):
    dout = b2.shape[1]
    return pl.pallas_call(
        _f_body,
        grid=(GRID,),
        in_specs=[
            pl.BlockSpec((1, BN, 128), lambda i: (0, i, 0)),
            pl.BlockSpec((1, BN, 128), lambda i: (1, i, 0)),
            pl.BlockSpec((BN, 128), lambda i: (i, 0)),
            pl.BlockSpec((BN, 1), lambda i: (i, 0)),
            pl.BlockSpec((1, dout), lambda i: (0, 0)),
        ],
        out_specs=pl.BlockSpec((BN, dout), lambda i: (i, 0)),
        out_shape=jax.ShapeDtypeStruct((N, dout), jnp.float32),
    )(acc2, acc2, hs2, dis, b2)


def kernel(x, edge_index, W1, b1, W2, b2):
    src = edge_index[0]
    dst = edge_index[1]
    pad = E_PAD - E
    src_p = jnp.concatenate([src, jnp.arange(pad, dtype=jnp.int32) % N])
    dump = N + jnp.arange(pad, dtype=jnp.int32) % (N_ACC - N)
    dst_p = jnp.concatenate([dst, dump])
    src2 = jnp.stack([src_p, src_p + N])

    zerosd = jnp.zeros((N_ACC,), jnp.float32)
    zeros1 = jnp.zeros((N_ACC, 128), jnp.float32)

    degout = _deg_call(dst_p, zerosd)
    degt = degout.T

    hs2n, dis = _b_call(x, W1, degt)
    tab1 = hs2n.reshape(2 * N, 128)
    acc1 = _agg_call(src2, dst_p, tab1, zeros1, True)

    hs2 = _d_call(acc1, hs2n, dis, b1.reshape(1, -1), W2)
    acc2 = _agg_call(src2, dst_p, hs2, zeros1, False)

    return _f_call(acc2, hs2, dis, b2.reshape(1, -1))
